# baseline (speedup 1.0000x reference)
import jax
import jax.numpy as jnp
from jax.experimental import pallas as pl
from jax.experimental.pallas import tpu as pltpu

TB = 2048


def _vq_block(x_ref, e_ref, xn2_ref, out_ref, ind_ref, es_ref, en2_ref):
    @pl.when(pl.program_id(0) == 0)
    def _():
        e = e_ref[...]
        es_ref[...] = e * (-2.0)
        en2_ref[...] = jnp.sum(e * e, axis=0, keepdims=True)

    x_blk = x_ref[...]
    es = es_ref[...]

    scores2 = jnp.dot(x_blk, es, preferred_element_type=jnp.float32)
    dist = (xn2_ref[...] + scores2) + en2_ref[...]
    K = dist.shape[1]
    mn = jnp.min(dist, axis=1, keepdims=True)
    k_iota = jax.lax.broadcasted_iota(jnp.int32, dist.shape, 1)
    idx = jnp.min(jnp.where(dist == mn, k_iota, K), axis=1).astype(jnp.int32)

    onehot = jnp.where(k_iota == idx[:, None], -0.5, 0.0)
    q = jax.lax.dot_general(
        onehot, es, (((1,), (1,)), ((), ())),
        preferred_element_type=jnp.float32)

    xx = jnp.sum(x_blk * x_blk, axis=1).reshape(TB // 128, 128)
    qq = jnp.sum(q * q, axis=1).reshape(TB // 128, 128)
    xq = jnp.sum(x_blk * q, axis=1).reshape(TB // 128, 128)

    e_inv = jnp.minimum(jax.lax.rsqrt(xx), 1e+06)
    q_inv = jnp.minimum(jax.lax.rsqrt(qq), 1e+06)
    e_norm = xx * e_inv
    q_norm = qq * q_inv
    lam = q_norm * e_inv
    c = xq * (e_inv * q_inv)
    ss = xx * (e_inv * e_inv) + 2.0 * c + qq * (q_inv * q_inv)
    ns_inv = jnp.minimum(jax.lax.rsqrt(ss), 1e+06)
    r_dot_e = (e_norm + xq * q_inv) * ns_inv
    a = lam * (1.0 - 2.0 * r_dot_e * ns_inv * e_inv)
    b = (lam * q_inv) * (2.0 * e_norm - 2.0 * r_dot_e * ns_inv)

    out_ref[...] = a.reshape(TB, 1) * x_blk + b.reshape(TB, 1) * q
    ind_ref[0, 0] = idx


@jax.jit
def kernel(x, e_i_ts):
    B, C, H, W = x.shape
    K = e_i_ts.shape[1]
    Ttot = B * H * W
    NB = Ttot // TB
    x_tok = jnp.transpose(x, (0, 2, 3, 1)).reshape(Ttot, C)
    xn2 = jnp.sum(x_tok ** 2, axis=1, keepdims=True)
    x_in = pltpu.with_memory_space_constraint(x_tok, pltpu.MemorySpace.HBM)

    out, ind = pl.pallas_call(
        _vq_block,
        grid=(NB,),
        in_specs=[
            pl.BlockSpec((TB, C), lambda t: (t, 0)),
            pl.BlockSpec((C, K), lambda t: (0, 0)),
            pl.BlockSpec((TB, 1), lambda t: (t, 0)),
        ],
        out_specs=[
            pl.BlockSpec((TB, C), lambda t: (t, 0)),
            pl.BlockSpec((1, 1, TB), lambda t: (t, 0, 0)),
        ],
        out_shape=[
            jax.ShapeDtypeStruct((Ttot, C), jnp.float32),
            jax.ShapeDtypeStruct((NB, 1, TB), jnp.int32),
        ],
        scratch_shapes=[
            pltpu.VMEM((C, K), jnp.float32),
            pltpu.VMEM((1, K), jnp.float32),
        ],
    )(x_in, e_i_ts, xn2)

    quant = jnp.transpose(out.reshape(B, H, W, C), (0, 3, 1, 2))
    return quant, ind.reshape(B, H, W)


# speedup vs baseline: 1.0544x; 1.0544x over previous
import jax
import jax.numpy as jnp
from jax.experimental import pallas as pl
from jax.experimental.pallas import tpu as pltpu

TB = 2048


def _vq_block(x_ref, e_ref, xn2_ref, out_ref, ind_ref, es_ref, en2_ref):
    @pl.when(pl.program_id(0) == 0)
    def _():
        e = e_ref[...]
        es_ref[...] = e * (-2.0)
        en2_ref[...] = jnp.sum(e * e, axis=0, keepdims=True)

    x_blk = x_ref[...]
    es = es_ref[...]

    scores2 = jnp.dot(x_blk, es, preferred_element_type=jnp.float32)
    dist = (xn2_ref[...] + scores2) + en2_ref[...]
    mn = jnp.min(dist, axis=1, keepdims=True)
    kf_iota = jax.lax.broadcasted_iota(jnp.int32, dist.shape, 1).astype(jnp.float32)
    idxf = jnp.min(jnp.where(dist == mn, kf_iota, 1024.0),
                   axis=1, keepdims=True)
    idx = idxf[:, 0].astype(jnp.int32)

    onehot = jnp.where(kf_iota == idxf, -0.5, 0.0)
    q = jax.lax.dot_general(
        onehot, es, (((1,), (1,)), ((), ())),
        preferred_element_type=jnp.float32)

    xx = jnp.sum(x_blk * x_blk, axis=1).reshape(TB // 128, 128)
    qq = jnp.sum(q * q, axis=1).reshape(TB // 128, 128)
    xq = jnp.sum(x_blk * q, axis=1).reshape(TB // 128, 128)

    e_inv = jnp.minimum(jax.lax.rsqrt(xx), 1e+06)
    q_inv = jnp.minimum(jax.lax.rsqrt(qq), 1e+06)
    e_norm = xx * e_inv
    q_norm = qq * q_inv
    lam = q_norm * e_inv
    c = xq * (e_inv * q_inv)
    ss = xx * (e_inv * e_inv) + 2.0 * c + qq * (q_inv * q_inv)
    ns_inv = jnp.minimum(jax.lax.rsqrt(ss), 1e+06)
    r_dot_e = (e_norm + xq * q_inv) * ns_inv
    a = lam * (1.0 - 2.0 * r_dot_e * ns_inv * e_inv)
    b = (lam * q_inv) * (2.0 * e_norm - 2.0 * r_dot_e * ns_inv)

    out_ref[...] = a.reshape(TB, 1) * x_blk + b.reshape(TB, 1) * q
    ind_ref[0, 0] = idx


@jax.jit
def kernel(x, e_i_ts):
    B, C, H, W = x.shape
    K = e_i_ts.shape[1]
    Ttot = B * H * W
    NB = Ttot // TB
    x_tok = jnp.transpose(x, (0, 2, 3, 1)).reshape(Ttot, C)
    xn2 = jnp.sum(x_tok ** 2, axis=1, keepdims=True)
    x_in = x_tok

    out, ind = pl.pallas_call(
        _vq_block,
        grid=(NB,),
        in_specs=[
            pl.BlockSpec((TB, C), lambda t: (t, 0)),
            pl.BlockSpec((C, K), lambda t: (0, 0)),
            pl.BlockSpec((TB, 1), lambda t: (t, 0)),
        ],
        out_specs=[
            pl.BlockSpec((TB, C), lambda t: (t, 0)),
            pl.BlockSpec((1, 1, TB), lambda t: (t, 0, 0)),
        ],
        out_shape=[
            jax.ShapeDtypeStruct((Ttot, C), jnp.float32),
            jax.ShapeDtypeStruct((NB, 1, TB), jnp.int32),
        ],
        scratch_shapes=[
            pltpu.VMEM((C, K), jnp.float32),
            pltpu.VMEM((1, K), jnp.float32),
        ],
    )(x_in, e_i_ts, xn2)

    quant = jnp.transpose(out.reshape(B, H, W, C), (0, 3, 1, 2))
    return quant, ind.reshape(B, H, W)
